# Initial kernel scaffold; baseline (speedup 1.0000x reference)
import jax
import jax.numpy as jnp
from jax.experimental import pallas as pl


def kernel(x, edge_index, edge_features_raw, same_type_mask, W_enc1, b_enc1, W_enc2, b_enc2, Wl_s, bl_s, Wr_s, br_s, We_s, att_s, bias_s, Wl_r, bl_r, Wr_r, br_r, We_r, att_r, bias_r):
    raise NotImplementedError("write your pallas kernel here")


# speedup vs baseline: 1.6033x; 1.6033x over previous
import functools

import jax
import jax.numpy as jnp
from jax import lax
from jax.experimental import pallas as pl
from jax.experimental.pallas import tpu as pltpu
from jax.experimental.pallas import tpu_sc as plsc

N = 10000
E = 320000
D = 128
EE = 16
NC = 2
NS = 16
NW = NC * NS
EPW = E // NW
B = 80
NB = EPW // B
TW = D + 16
NPAD = 10240
RPT = NPAD // NS


def _node_proj_body(x, wls, bls, wrs, brs, wlr, blr, wrr, brr,
                    xls, xrs, xlr, xrr):
    xb = x[...]
    xls[...] = jnp.dot(xb, wls[...], preferred_element_type=jnp.float32) + bls[...]
    xrs[...] = jnp.dot(xb, wrs[...], preferred_element_type=jnp.float32) + brs[...]
    xlr[...] = jnp.dot(xb, wlr[...], preferred_element_type=jnp.float32) + blr[...]
    xrr[...] = jnp.dot(xb, wrr[...], preferred_element_type=jnp.float32) + brr[...]


def _edge_enc_body(raw, w1, b1, w2, b2, wes, wer, ews, ewr):
    h = jnp.maximum(jnp.dot(raw[...], w1[...],
                            preferred_element_type=jnp.float32) + b1[...], 0.0)
    ea = jnp.dot(h, w2[...], preferred_element_type=jnp.float32) + b2[...]
    ews[...] = jnp.dot(ea, wes[...], preferred_element_type=jnp.float32)
    ewr[...] = jnp.dot(ea, wer[...], preferred_element_type=jnp.float32)


def _finalize_body(outm, outd, bias, o):
    p = outm[...]
    q = outd[...]
    s = p[0, 0] + p[0, 1]
    r = p[1, 0] + p[1, 1]
    ds_ = q[0, 0] + q[0, 1]
    dr_ = q[1, 0] + q[1, 1]
    os_ = s / (ds_[:, 0:1] + 1e-16)
    or_ = r / (dr_[:, 0:1] + 1e-16)
    o[...] = jnp.concatenate([os_, or_], axis=-1) + bias[...]


def _make_sc_edge_pass(npad, b, nb, rpt, interpret=False):
    mesh = plsc.VectorSubcoreMesh(core_axis_name="c", subcore_axis_name="s")

    @functools.partial(
        pl.kernel,
        mesh=mesh,
        interpret=interpret,
        compiler_params=pltpu.CompilerParams(use_tc_tiling_on_sc=False,
                                             needs_layout_passes=False),
        out_type=(jax.ShapeDtypeStruct((2, NC, npad, D), jnp.float32),
                  jax.ShapeDtypeStruct((2, NC, npad, 16), jnp.float32)),
        scratch_types=[
            pltpu.VMEM((b,), jnp.int32),
            pltpu.VMEM((b,), jnp.int32),
            pltpu.VMEM((b,), jnp.float32),
            pltpu.VMEM((2, D), jnp.float32),
            pltpu.VMEM((b, D), jnp.float32),
            pltpu.VMEM((b, D), jnp.float32),
            pltpu.VMEM((b, D), jnp.float32),
            pltpu.VMEM((b, 16), jnp.float32),
            pltpu.VMEM_SHARED((npad, D), jnp.float32),
            pltpu.VMEM_SHARED((npad, 16), jnp.float32),
        ],
    )
    def _sc_edge_pass(src_h, dst_h, mask_h, att_h, xls_h, xrs_h, ews_h,
                      xlr_h, xrr_h, ewr_h, zerom_h, zerod_h, outm_h, outd_h,
                      src_v, dst_v, mask_v, att_v, xl_v, xr_v, ew_v, exr_v,
                      accm_sh, accd_sh):
        cid = lax.axis_index("c")
        sid = lax.axis_index("s")
        wid = sid * NC + cid

        pltpu.sync_copy(att_h, att_v)
        for e in range(b):
            exr_v[e, :] = jnp.zeros((16,), jnp.float32)

        def run_head(h, xl_t, xr_t, ew_t):
            pltpu.sync_copy(zerom_h, accm_sh.at[pl.ds(sid * rpt, rpt)])
            pltpu.sync_copy(zerod_h, accd_sh.at[pl.ds(sid * rpt, rpt)])
            plsc.subcore_barrier()

            def batch_body(j, carry):
                pltpu.sync_copy(src_h.at[wid, j], src_v)
                pltpu.sync_copy(dst_h.at[wid, j], dst_v)
                if h == 1:
                    pltpu.sync_copy(mask_h.at[wid, j], mask_v)
                pltpu.sync_copy(xl_t.at[src_v], xl_v)
                pltpu.sync_copy(xr_t.at[dst_v], xr_v)
                pltpu.sync_copy(ew_t.at[pl.ds((wid * nb + j) * b, b)], ew_v)

                def group_body(g, carry2):
                    rows16 = g * 16 + lax.iota(jnp.int32, 16)

                    def kblock(k, acc):
                        for cc in range(16):
                            c = k * 16 + cc
                            colv = jnp.full((16,), c, jnp.int32)
                            xlv = plsc.load_gather(xl_v, [rows16, colv])
                            xrv = plsc.load_gather(xr_v, [rows16, colv])
                            ewv = plsc.load_gather(ew_v, [rows16, colv])
                            attv = plsc.load_gather(
                                att_v, [jnp.full((16,), h, jnp.int32), colv])
                            m = xlv + xrv + ewv
                            m = jnp.maximum(m, 0.2 * m)
                            acc = acc + attv * m
                        return acc

                    alpha = lax.fori_loop(0, 8, kblock,
                                          jnp.zeros((16,), jnp.float32))
                    exv = jnp.exp(alpha)
                    if h == 1:
                        exv = exv * mask_v[pl.ds(g * 16, 16)]
                    plsc.store_scatter(exr_v,
                                       [rows16, jnp.zeros((16,), jnp.int32)],
                                       exv)

                    def kscale(k, carry3):
                        for cc in range(16):
                            c = k * 16 + cc
                            colv = jnp.full((16,), c, jnp.int32)
                            v = plsc.load_gather(xl_v, [rows16, colv]) * exv
                            plsc.store_scatter(xl_v, [rows16, colv], v)
                        return carry3

                    lax.fori_loop(0, 8, kscale, 0)
                    return carry2

                lax.fori_loop(0, b // 16, group_body, 0)
                pltpu.sync_copy(xl_v, accm_sh.at[dst_v], add=True)
                pltpu.sync_copy(exr_v, accd_sh.at[dst_v], add=True)
                return carry

            lax.fori_loop(0, nb, batch_body, 0)
            plsc.subcore_barrier()
            pltpu.sync_copy(accm_sh.at[pl.ds(sid * rpt, rpt)],
                            outm_h.at[h, cid, pl.ds(sid * rpt, rpt)])
            pltpu.sync_copy(accd_sh.at[pl.ds(sid * rpt, rpt)],
                            outd_h.at[h, cid, pl.ds(sid * rpt, rpt)])
            plsc.subcore_barrier()

        run_head(0, xls_h, xrs_h, ews_h)
        run_head(1, xlr_h, xrr_h, ewr_h)

    return _sc_edge_pass


_sc_edge_pass = _make_sc_edge_pass(NPAD, B, NB, RPT)


def _node_proj(x, Wl_s, bl_s, Wr_s, br_s, Wl_r, bl_r, Wr_r, br_r):
    f32 = jnp.float32
    blk_n = 2000
    return pl.pallas_call(
        _node_proj_body,
        grid=(N // blk_n,),
        in_specs=[pl.BlockSpec((blk_n, D), lambda i: (i, 0))]
        + [pl.BlockSpec((D, D), lambda i: (0, 0)),
           pl.BlockSpec((1, D), lambda i: (0, 0))] * 4,
        out_specs=[pl.BlockSpec((blk_n, D), lambda i: (i, 0))] * 4,
        out_shape=[jax.ShapeDtypeStruct((N, D), f32)] * 4,
    )(x, Wl_s, bl_s.reshape(1, D), Wr_s, br_s.reshape(1, D),
      Wl_r, bl_r.reshape(1, D), Wr_r, br_r.reshape(1, D))


def _edge_enc(edge_features_raw, W_enc1, b_enc1, W_enc2, b_enc2, We_s, We_r):
    f32 = jnp.float32
    raw8 = jnp.pad(edge_features_raw, ((0, 0), (0, 1)))
    w1p = jnp.pad(W_enc1, ((0, 1), (0, 0)))
    blk_e = 4000
    return pl.pallas_call(
        _edge_enc_body,
        grid=(E // blk_e,),
        in_specs=[pl.BlockSpec((blk_e, 8), lambda i: (i, 0)),
                  pl.BlockSpec((8, 32), lambda i: (0, 0)),
                  pl.BlockSpec((1, 32), lambda i: (0, 0)),
                  pl.BlockSpec((32, EE), lambda i: (0, 0)),
                  pl.BlockSpec((1, EE), lambda i: (0, 0)),
                  pl.BlockSpec((EE, D), lambda i: (0, 0)),
                  pl.BlockSpec((EE, D), lambda i: (0, 0))],
        out_specs=[pl.BlockSpec((blk_e, D), lambda i: (i, 0))] * 2,
        out_shape=[jax.ShapeDtypeStruct((E, D), f32)] * 2,
    )(raw8, w1p, b_enc1.reshape(1, 32), W_enc2, b_enc2.reshape(1, EE),
      We_s, We_r)


def _finalize(outm, outd, bias_full):
    f32 = jnp.float32
    blk_f = 2000
    return pl.pallas_call(
        _finalize_body,
        grid=(N // blk_f,),
        in_specs=[pl.BlockSpec((2, NC, blk_f, D), lambda i: (0, 0, i, 0)),
                  pl.BlockSpec((2, NC, blk_f, 16), lambda i: (0, 0, i, 0)),
                  pl.BlockSpec((1, 2 * D), lambda i: (0, 0))],
        out_specs=pl.BlockSpec((blk_f, 2 * D), lambda i: (i, 0)),
        out_shape=jax.ShapeDtypeStruct((N, 2 * D), f32),
    )(outm, outd, bias_full)


def kernel(x, edge_index, edge_features_raw, same_type_mask,
           W_enc1, b_enc1, W_enc2, b_enc2,
           Wl_s, bl_s, Wr_s, br_s, We_s, att_s, bias_s,
           Wl_r, bl_r, Wr_r, br_r, We_r, att_r, bias_r):
    f32 = jnp.float32
    xls, xrs, xlr, xrr = _node_proj(x, Wl_s, bl_s, Wr_s, br_s,
                                    Wl_r, bl_r, Wr_r, br_r)
    ews, ewr = _edge_enc(edge_features_raw, W_enc1, b_enc1, W_enc2, b_enc2,
                         We_s, We_r)

    src2 = edge_index[0].astype(jnp.int32).reshape(NW, NB, B)
    dst2 = edge_index[1].astype(jnp.int32).reshape(NW, NB, B)
    mask2 = same_type_mask.astype(f32).reshape(NW, NB, B)
    att2 = jnp.concatenate([att_s.reshape(1, D), att_r.reshape(1, D)], axis=0)
    zerom = jnp.zeros((RPT, D), f32)
    zerod = jnp.zeros((RPT, 16), f32)
    outm, outd = _sc_edge_pass(src2, dst2, mask2, att2, xls, xrs, ews,
                               xlr, xrr, ewr, zerom, zerod)

    bias_full = jnp.concatenate([bias_s, bias_r]).reshape(1, 2 * D)
    return _finalize(outm, outd, bias_full)


# speedup vs baseline: 1.7363x; 1.0829x over previous
import functools

import jax
import jax.numpy as jnp
from jax import lax
from jax.experimental import pallas as pl
from jax.experimental.pallas import tpu as pltpu
from jax.experimental.pallas import tpu_sc as plsc

N = 10000
E = 320000
D = 128
EE = 16
NC = 2
NS = 16
NW = NC * NS
EPW = E // NW
B = 80
NB = EPW // B
TW = D + 16
NPAD = 10240
RPT = NPAD // NS


def _node_proj_body(x, wls, bls, wrs, brs, wlr, blr, wrr, brr,
                    xls, xrs, xlr, xrr):
    xb = x[...]
    xls[...] = jnp.dot(xb, wls[...], preferred_element_type=jnp.float32) + bls[...]
    xrs[...] = jnp.dot(xb, wrs[...], preferred_element_type=jnp.float32) + brs[...]
    xlr[...] = jnp.dot(xb, wlr[...], preferred_element_type=jnp.float32) + blr[...]
    xrr[...] = jnp.dot(xb, wrr[...], preferred_element_type=jnp.float32) + brr[...]


def _edge_enc_body(raw, w1, b1, w2, b2, wes, wer, ews, ewr):
    h = jnp.maximum(jnp.dot(raw[...], w1[...],
                            preferred_element_type=jnp.float32) + b1[...], 0.0)
    ea = jnp.dot(h, w2[...], preferred_element_type=jnp.float32) + b2[...]
    ews[...] = jnp.dot(ea, wes[...], preferred_element_type=jnp.float32)
    ewr[...] = jnp.dot(ea, wer[...], preferred_element_type=jnp.float32)


def _finalize_body(outm, outd, bias, o):
    p = outm[...]
    q = outd[...]
    s = p[0, 0] + p[0, 1]
    r = p[1, 0] + p[1, 1]
    ds_ = q[0, 0] + q[0, 1]
    dr_ = q[1, 0] + q[1, 1]
    os_ = s / (ds_[:, 0:1] + 1e-16)
    or_ = r / (dr_[:, 0:1] + 1e-16)
    o[...] = jnp.concatenate([os_, or_], axis=-1) + bias[...]


def _make_sc_edge_pass(npad, b, nb, rpt, interpret=False):
    mesh = plsc.VectorSubcoreMesh(core_axis_name="c", subcore_axis_name="s")

    @functools.partial(
        pl.kernel,
        mesh=mesh,
        interpret=interpret,
        compiler_params=pltpu.CompilerParams(use_tc_tiling_on_sc=False,
                                             needs_layout_passes=False),
        out_type=(jax.ShapeDtypeStruct((2, NC, npad, D), jnp.float32),
                  jax.ShapeDtypeStruct((2, NC, npad, 16), jnp.float32)),
        scratch_types=[
            pltpu.VMEM((2, b), jnp.int32),
            pltpu.VMEM((2, b), jnp.int32),
            pltpu.VMEM((2, b), jnp.float32),
            pltpu.VMEM((2, D), jnp.float32),
            pltpu.VMEM((b, D), jnp.float32),
            pltpu.VMEM((b, D), jnp.float32),
            pltpu.VMEM((b, D), jnp.float32),
            pltpu.VMEM((b, 16), jnp.float32),
            pltpu.VMEM_SHARED((npad, D), jnp.float32),
            pltpu.VMEM_SHARED((npad, 16), jnp.float32),
            pltpu.SemaphoreType.DMA,
            pltpu.SemaphoreType.DMA,
            pltpu.SemaphoreType.DMA,
            pltpu.SemaphoreType.DMA,
        ],
    )
    def _sc_edge_pass(src_h, dst_h, mask_h, att_h, xls_h, xrs_h, ews_h,
                      xlr_h, xrr_h, ewr_h, zerom_h, zerod_h, outm_h, outd_h,
                      src_v, dst_v, mask_v, att_v, xl_v, xr_v, ew_v, exr_v,
                      accm_sh, accd_sh, sem_i, sem_g, sem_sm, sem_sd):
        cid = lax.axis_index("c")
        sid = lax.axis_index("s")
        wid = sid * NC + cid

        pltpu.sync_copy(att_h, att_v)
        for e in range(b):
            exr_v[e, :] = jnp.zeros((16,), jnp.float32)

        def run_head(h, xl_t, xr_t, ew_t):
            pltpu.sync_copy(zerom_h, accm_sh.at[pl.ds(sid * rpt, rpt)])
            pltpu.sync_copy(zerod_h, accd_sh.at[pl.ds(sid * rpt, rpt)])
            plsc.subcore_barrier()

            def issue_idx(j, slot):
                pltpu.async_copy(src_h.at[wid, j], src_v.at[slot], sem_i)
                pltpu.async_copy(dst_h.at[wid, j], dst_v.at[slot], sem_i)
                if h == 1:
                    pltpu.async_copy(mask_h.at[wid, j], mask_v.at[slot], sem_i)

            def wait_idx():
                pltpu.make_async_copy(src_h.at[wid, 0], src_v.at[0], sem_i).wait()
                pltpu.make_async_copy(dst_h.at[wid, 0], dst_v.at[0], sem_i).wait()
                if h == 1:
                    pltpu.make_async_copy(mask_h.at[wid, 0], mask_v.at[0],
                                          sem_i).wait()

            issue_idx(0, 0)

            def batch_body(j, carry):
                jm = lax.rem(j, 2)
                wait_idx()

                @pl.when(j > 0)
                def _():
                    pltpu.make_async_copy(xl_v, accm_sh.at[dst_v.at[1 - jm]],
                                          sem_sm).wait()
                    pltpu.make_async_copy(exr_v, accd_sh.at[dst_v.at[1 - jm]],
                                          sem_sd).wait()

                d1 = pltpu.async_copy(xl_t.at[src_v.at[jm]], xl_v, sem_g)
                d2 = pltpu.async_copy(xr_t.at[dst_v.at[jm]], xr_v, sem_g)
                d3 = pltpu.async_copy(ew_t.at[pl.ds((wid * nb + j) * b, b)],
                                      ew_v, sem_g)

                @pl.when(j + 1 < nb)
                def _():
                    issue_idx(j + 1, 1 - jm)

                d1.wait()
                d2.wait()
                d3.wait()

                def group_body(g, carry2):
                    rows16 = g * 16 + lax.iota(jnp.int32, 16)

                    def kblock(k, acc):
                        for cc in range(16):
                            c = k * 16 + cc
                            colv = jnp.full((16,), c, jnp.int32)
                            xlv = plsc.load_gather(xl_v, [rows16, colv])
                            xrv = plsc.load_gather(xr_v, [rows16, colv])
                            ewv = plsc.load_gather(ew_v, [rows16, colv])
                            attv = plsc.load_gather(
                                att_v, [jnp.full((16,), h, jnp.int32), colv])
                            m = xlv + xrv + ewv
                            m = jnp.maximum(m, 0.2 * m)
                            acc = acc + attv * m
                        return acc

                    alpha = lax.fori_loop(0, 8, kblock,
                                          jnp.zeros((16,), jnp.float32))
                    exv = jnp.exp(alpha)
                    if h == 1:
                        exv = exv * mask_v[jm, pl.ds(g * 16, 16)]
                    plsc.store_scatter(exr_v,
                                       [rows16, jnp.zeros((16,), jnp.int32)],
                                       exv)

                    def kscale(k, carry3):
                        for cc in range(16):
                            c = k * 16 + cc
                            colv = jnp.full((16,), c, jnp.int32)
                            v = plsc.load_gather(xl_v, [rows16, colv]) * exv
                            plsc.store_scatter(xl_v, [rows16, colv], v)
                        return carry3

                    lax.fori_loop(0, 8, kscale, 0)
                    return carry2

                lax.fori_loop(0, b // 16, group_body, 0)
                pltpu.async_copy(xl_v, accm_sh.at[dst_v.at[jm]], sem_sm,
                                 add=True)
                pltpu.async_copy(exr_v, accd_sh.at[dst_v.at[jm]], sem_sd,
                                 add=True)
                return carry

            lax.fori_loop(0, nb, batch_body, 0)
            last = (nb - 1) % 2
            pltpu.make_async_copy(xl_v, accm_sh.at[dst_v.at[last]],
                                  sem_sm).wait()
            pltpu.make_async_copy(exr_v, accd_sh.at[dst_v.at[last]],
                                  sem_sd).wait()
            plsc.subcore_barrier()
            pltpu.sync_copy(accm_sh.at[pl.ds(sid * rpt, rpt)],
                            outm_h.at[h, cid, pl.ds(sid * rpt, rpt)])
            pltpu.sync_copy(accd_sh.at[pl.ds(sid * rpt, rpt)],
                            outd_h.at[h, cid, pl.ds(sid * rpt, rpt)])
            plsc.subcore_barrier()

        run_head(0, xls_h, xrs_h, ews_h)
        run_head(1, xlr_h, xrr_h, ewr_h)

    return _sc_edge_pass


_sc_edge_pass = _make_sc_edge_pass(NPAD, B, NB, RPT)


def _node_proj(x, Wl_s, bl_s, Wr_s, br_s, Wl_r, bl_r, Wr_r, br_r):
    f32 = jnp.float32
    blk_n = 2000
    return pl.pallas_call(
        _node_proj_body,
        grid=(N // blk_n,),
        in_specs=[pl.BlockSpec((blk_n, D), lambda i: (i, 0))]
        + [pl.BlockSpec((D, D), lambda i: (0, 0)),
           pl.BlockSpec((1, D), lambda i: (0, 0))] * 4,
        out_specs=[pl.BlockSpec((blk_n, D), lambda i: (i, 0))] * 4,
        out_shape=[jax.ShapeDtypeStruct((N, D), f32)] * 4,
    )(x, Wl_s, bl_s.reshape(1, D), Wr_s, br_s.reshape(1, D),
      Wl_r, bl_r.reshape(1, D), Wr_r, br_r.reshape(1, D))


def _edge_enc(edge_features_raw, W_enc1, b_enc1, W_enc2, b_enc2, We_s, We_r):
    f32 = jnp.float32
    raw8 = jnp.pad(edge_features_raw, ((0, 0), (0, 1)))
    w1p = jnp.pad(W_enc1, ((0, 1), (0, 0)))
    blk_e = 4000
    return pl.pallas_call(
        _edge_enc_body,
        grid=(E // blk_e,),
        in_specs=[pl.BlockSpec((blk_e, 8), lambda i: (i, 0)),
                  pl.BlockSpec((8, 32), lambda i: (0, 0)),
                  pl.BlockSpec((1, 32), lambda i: (0, 0)),
                  pl.BlockSpec((32, EE), lambda i: (0, 0)),
                  pl.BlockSpec((1, EE), lambda i: (0, 0)),
                  pl.BlockSpec((EE, D), lambda i: (0, 0)),
                  pl.BlockSpec((EE, D), lambda i: (0, 0))],
        out_specs=[pl.BlockSpec((blk_e, D), lambda i: (i, 0))] * 2,
        out_shape=[jax.ShapeDtypeStruct((E, D), f32)] * 2,
    )(raw8, w1p, b_enc1.reshape(1, 32), W_enc2, b_enc2.reshape(1, EE),
      We_s, We_r)


def _finalize(outm, outd, bias_full):
    f32 = jnp.float32
    blk_f = 2000
    return pl.pallas_call(
        _finalize_body,
        grid=(N // blk_f,),
        in_specs=[pl.BlockSpec((2, NC, blk_f, D), lambda i: (0, 0, i, 0)),
                  pl.BlockSpec((2, NC, blk_f, 16), lambda i: (0, 0, i, 0)),
                  pl.BlockSpec((1, 2 * D), lambda i: (0, 0))],
        out_specs=pl.BlockSpec((blk_f, 2 * D), lambda i: (i, 0)),
        out_shape=jax.ShapeDtypeStruct((N, 2 * D), f32),
    )(outm, outd, bias_full)


def kernel(x, edge_index, edge_features_raw, same_type_mask,
           W_enc1, b_enc1, W_enc2, b_enc2,
           Wl_s, bl_s, Wr_s, br_s, We_s, att_s, bias_s,
           Wl_r, bl_r, Wr_r, br_r, We_r, att_r, bias_r):
    f32 = jnp.float32
    xls, xrs, xlr, xrr = _node_proj(x, Wl_s, bl_s, Wr_s, br_s,
                                    Wl_r, bl_r, Wr_r, br_r)
    ews, ewr = _edge_enc(edge_features_raw, W_enc1, b_enc1, W_enc2, b_enc2,
                         We_s, We_r)

    src2 = edge_index[0].astype(jnp.int32).reshape(NW, NB, B)
    dst2 = edge_index[1].astype(jnp.int32).reshape(NW, NB, B)
    mask2 = same_type_mask.astype(f32).reshape(NW, NB, B)
    att2 = jnp.concatenate([att_s.reshape(1, D), att_r.reshape(1, D)], axis=0)
    zerom = jnp.zeros((RPT, D), f32)
    zerod = jnp.zeros((RPT, 16), f32)
    outm, outd = _sc_edge_pass(src2, dst2, mask2, att2, xls, xrs, ews,
                               xlr, xrr, ewr, zerom, zerod)

    bias_full = jnp.concatenate([bias_s, bias_r]).reshape(1, 2 * D)
    return _finalize(outm, outd, bias_full)


# speedup vs baseline: 2.1820x; 1.2567x over previous
import functools

import jax
import jax.numpy as jnp
from jax import lax
from jax.experimental import pallas as pl
from jax.experimental.pallas import tpu as pltpu
from jax.experimental.pallas import tpu_sc as plsc

N = 10000
E = 320000
D = 128
EE = 16
NC = 2
NS = 16
NW = NC * NS
EPW = E // NW
B = 80
NB = EPW // B
TW = D + 16
NPAD = 10240
RPT = NPAD // NS


def _node_proj_body(x, wls, bls, wrs, brs, wlr, blr, wrr, brr,
                    xls, xrs, xlr, xrr):
    xb = x[...]
    xls[...] = jnp.dot(xb, wls[...], preferred_element_type=jnp.float32) + bls[...]
    xrs[...] = jnp.dot(xb, wrs[...], preferred_element_type=jnp.float32) + brs[...]
    xlr[...] = jnp.dot(xb, wlr[...], preferred_element_type=jnp.float32) + blr[...]
    xrr[...] = jnp.dot(xb, wrr[...], preferred_element_type=jnp.float32) + brr[...]


def _edge_enc_body(raw, w1, b1, w2, b2, wes, wer, ews, ewr):
    h = jnp.maximum(jnp.dot(raw[...], w1[...],
                            preferred_element_type=jnp.float32) + b1[...], 0.0)
    ea = jnp.dot(h, w2[...], preferred_element_type=jnp.float32) + b2[...]
    ews[...] = jnp.dot(ea, wes[...], preferred_element_type=jnp.float32)
    ewr[...] = jnp.dot(ea, wer[...], preferred_element_type=jnp.float32)


def _finalize_body(outm, bias, o):
    p = outm[...]
    s = p[0, 0] + p[0, 1]
    r = p[1, 0] + p[1, 1]
    os_ = s[:, :D] / (s[:, D:D + 1] + 1e-16)
    or_ = r[:, :D] / (r[:, D:D + 1] + 1e-16)
    o[...] = jnp.concatenate([os_, or_], axis=-1) + bias[...]


def _make_sc_edge_pass(npad, b, nb, rpt, interpret=False):
    mesh = plsc.VectorSubcoreMesh(core_axis_name="c", subcore_axis_name="s")

    @functools.partial(
        pl.kernel,
        mesh=mesh,
        interpret=interpret,
        compiler_params=pltpu.CompilerParams(use_tc_tiling_on_sc=False,
                                             needs_layout_passes=False),
        out_type=jax.ShapeDtypeStruct((2, NC, npad, TW), jnp.float32),
        scratch_types=[
            pltpu.VMEM((2, b), jnp.int32),
            pltpu.VMEM((2, b), jnp.int32),
            pltpu.VMEM((2, b), jnp.float32),
            pltpu.VMEM((2, D), jnp.float32),
            pltpu.VMEM((b, D), jnp.float32),
            pltpu.VMEM((b, D), jnp.float32),
            pltpu.VMEM((b, TW), jnp.float32),
            pltpu.VMEM_SHARED((npad, TW), jnp.float32),
            pltpu.SemaphoreType.DMA,
            pltpu.SemaphoreType.DMA,
            pltpu.SemaphoreType.DMA,
        ],
    )
    def _sc_edge_pass(src_h, dst_h, mask_h, att_h, xls_h, xrs_h, ews_h,
                      xlr_h, xrr_h, ewr_h, zerom_h, outm_h,
                      src_v, dst_v, mask_v, att_v, xl_v, xr_v, msg_v,
                      accm_sh, sem_i, sem_g, sem_sm):
        cid = lax.axis_index("c")
        sid = lax.axis_index("s")
        wid = sid * NC + cid

        pltpu.sync_copy(att_h, att_v)
        for e in range(b):
            msg_v[e, pl.ds(D, 16)] = jnp.zeros((16,), jnp.float32)

        def run_head(h, xl_t, xr_t, ew_t):
            pltpu.sync_copy(zerom_h, accm_sh.at[pl.ds(sid * rpt, rpt)])
            plsc.subcore_barrier()

            def issue_idx(j, slot):
                pltpu.async_copy(src_h.at[wid, j], src_v.at[slot], sem_i)
                pltpu.async_copy(dst_h.at[wid, j], dst_v.at[slot], sem_i)
                if h == 1:
                    pltpu.async_copy(mask_h.at[wid, j], mask_v.at[slot], sem_i)

            def wait_idx():
                pltpu.make_async_copy(src_h.at[wid, 0], src_v.at[0], sem_i).wait()
                pltpu.make_async_copy(dst_h.at[wid, 0], dst_v.at[0], sem_i).wait()
                if h == 1:
                    pltpu.make_async_copy(mask_h.at[wid, 0], mask_v.at[0],
                                          sem_i).wait()

            issue_idx(0, 0)

            def batch_body(j, carry):
                jm = lax.rem(j, 2)
                wait_idx()

                @pl.when(j > 0)
                def _():
                    pltpu.make_async_copy(msg_v, accm_sh.at[dst_v.at[1 - jm]],
                                          sem_sm).wait()

                d1 = pltpu.async_copy(xl_t.at[src_v.at[jm]], xl_v, sem_g)
                d2 = pltpu.async_copy(xr_t.at[dst_v.at[jm]], xr_v, sem_g)
                d3 = pltpu.async_copy(ew_t.at[pl.ds((wid * nb + j) * b, b)],
                                      msg_v.at[:, pl.ds(0, D)], sem_g)

                @pl.when(j + 1 < nb)
                def _():
                    issue_idx(j + 1, 1 - jm)

                d1.wait()
                d2.wait()
                d3.wait()

                def group_body(g, carry2):
                    rows16 = g * 16 + lax.iota(jnp.int32, 16)

                    def kblock(k, acc):
                        for cc in range(16):
                            c = k * 16 + cc
                            colv = jnp.full((16,), c, jnp.int32)
                            xlv = plsc.load_gather(xl_v, [rows16, colv])
                            xrv = plsc.load_gather(xr_v, [rows16, colv])
                            ewv = plsc.load_gather(msg_v, [rows16, colv])
                            attv = plsc.load_gather(
                                att_v, [jnp.full((16,), h, jnp.int32), colv])
                            m = xlv + xrv + ewv
                            m = jnp.maximum(m, 0.2 * m)
                            acc = acc + attv * m
                        return acc

                    alpha = lax.fori_loop(0, 8, kblock,
                                          jnp.zeros((16,), jnp.float32))
                    exv = jnp.exp(alpha)
                    if h == 1:
                        exv = exv * mask_v[jm, pl.ds(g * 16, 16)]
                    plsc.store_scatter(msg_v,
                                       [rows16, jnp.full((16,), D, jnp.int32)],
                                       exv)

                    def kscale(k, carry3):
                        for cc in range(16):
                            c = k * 16 + cc
                            colv = jnp.full((16,), c, jnp.int32)
                            v = plsc.load_gather(xl_v, [rows16, colv]) * exv
                            plsc.store_scatter(msg_v, [rows16, colv], v)
                        return carry3

                    lax.fori_loop(0, 8, kscale, 0)
                    return carry2

                lax.fori_loop(0, b // 16, group_body, 0)
                pltpu.async_copy(msg_v, accm_sh.at[dst_v.at[jm]], sem_sm,
                                 add=True)
                return carry

            lax.fori_loop(0, nb, batch_body, 0)
            last = (nb - 1) % 2
            pltpu.make_async_copy(msg_v, accm_sh.at[dst_v.at[last]],
                                  sem_sm).wait()
            plsc.subcore_barrier()
            pltpu.sync_copy(accm_sh.at[pl.ds(sid * rpt, rpt)],
                            outm_h.at[h, cid, pl.ds(sid * rpt, rpt)])
            plsc.subcore_barrier()

        run_head(0, xls_h, xrs_h, ews_h)
        run_head(1, xlr_h, xrr_h, ewr_h)

    return _sc_edge_pass


_sc_edge_pass = _make_sc_edge_pass(NPAD, B, NB, RPT)


def _node_proj(x, Wl_s, bl_s, Wr_s, br_s, Wl_r, bl_r, Wr_r, br_r):
    f32 = jnp.float32
    blk_n = 2000
    return pl.pallas_call(
        _node_proj_body,
        grid=(N // blk_n,),
        in_specs=[pl.BlockSpec((blk_n, D), lambda i: (i, 0))]
        + [pl.BlockSpec((D, D), lambda i: (0, 0)),
           pl.BlockSpec((1, D), lambda i: (0, 0))] * 4,
        out_specs=[pl.BlockSpec((blk_n, D), lambda i: (i, 0))] * 4,
        out_shape=[jax.ShapeDtypeStruct((N, D), f32)] * 4,
    )(x, Wl_s, bl_s.reshape(1, D), Wr_s, br_s.reshape(1, D),
      Wl_r, bl_r.reshape(1, D), Wr_r, br_r.reshape(1, D))


def _edge_enc(edge_features_raw, W_enc1, b_enc1, W_enc2, b_enc2, We_s, We_r):
    f32 = jnp.float32
    raw8 = jnp.pad(edge_features_raw, ((0, 0), (0, 1)))
    w1p = jnp.pad(W_enc1, ((0, 1), (0, 0)))
    blk_e = 4000
    return pl.pallas_call(
        _edge_enc_body,
        grid=(E // blk_e,),
        in_specs=[pl.BlockSpec((blk_e, 8), lambda i: (i, 0)),
                  pl.BlockSpec((8, 32), lambda i: (0, 0)),
                  pl.BlockSpec((1, 32), lambda i: (0, 0)),
                  pl.BlockSpec((32, EE), lambda i: (0, 0)),
                  pl.BlockSpec((1, EE), lambda i: (0, 0)),
                  pl.BlockSpec((EE, D), lambda i: (0, 0)),
                  pl.BlockSpec((EE, D), lambda i: (0, 0))],
        out_specs=[pl.BlockSpec((blk_e, D), lambda i: (i, 0))] * 2,
        out_shape=[jax.ShapeDtypeStruct((E, D), f32)] * 2,
    )(raw8, w1p, b_enc1.reshape(1, 32), W_enc2, b_enc2.reshape(1, EE),
      We_s, We_r)


def _finalize(outm, bias_full):
    f32 = jnp.float32
    blk_f = 2000
    return pl.pallas_call(
        _finalize_body,
        grid=(N // blk_f,),
        in_specs=[pl.BlockSpec((2, NC, blk_f, TW), lambda i: (0, 0, i, 0)),
                  pl.BlockSpec((1, 2 * D), lambda i: (0, 0))],
        out_specs=pl.BlockSpec((blk_f, 2 * D), lambda i: (i, 0)),
        out_shape=jax.ShapeDtypeStruct((N, 2 * D), f32),
    )(outm, bias_full)


def kernel(x, edge_index, edge_features_raw, same_type_mask,
           W_enc1, b_enc1, W_enc2, b_enc2,
           Wl_s, bl_s, Wr_s, br_s, We_s, att_s, bias_s,
           Wl_r, bl_r, Wr_r, br_r, We_r, att_r, bias_r):
    f32 = jnp.float32
    xls, xrs, xlr, xrr = _node_proj(x, Wl_s, bl_s, Wr_s, br_s,
                                    Wl_r, bl_r, Wr_r, br_r)
    ews, ewr = _edge_enc(edge_features_raw, W_enc1, b_enc1, W_enc2, b_enc2,
                         We_s, We_r)

    src2 = edge_index[0].astype(jnp.int32).reshape(NW, NB, B)
    dst2 = edge_index[1].astype(jnp.int32).reshape(NW, NB, B)
    mask2 = same_type_mask.astype(f32).reshape(NW, NB, B)
    att2 = jnp.concatenate([att_s.reshape(1, D), att_r.reshape(1, D)], axis=0)
    zerom = jnp.zeros((RPT, TW), f32)
    outm = _sc_edge_pass(src2, dst2, mask2, att2, xls, xrs, ews,
                         xlr, xrr, ewr, zerom)

    bias_full = jnp.concatenate([bias_s, bias_r]).reshape(1, 2 * D)
    return _finalize(outm, bias_full)
